# baseline (speedup 1.0000x reference)
import dataclasses
import functools

import jax
import jax.numpy as jnp
from jax import lax
from jax.experimental import pallas as pl
from jax.experimental.pallas import tpu as pltpu
from jax.experimental.pallas import tpu_sc as plsc

T = 2048
D = 768
E = 8
FF = 512
BM = 256
CAP = 6144
NB = CAP // BM
SB = 512

NWORK = 32
CHD = 64
CHC = 32


def _gate_route_body(x_ref, gw_ref, d0_ref, d1_ref, be_ref, wb0_ref, wb1_ref,
                     xp_ref):
    x = x_ref[...]
    gw = gw_ref[...]
    logits = lax.dot_general(x, gw, (((1,), (1,)), ((), ())),
                             preferred_element_type=jnp.float32)

    m1 = jnp.max(logits, axis=1, keepdims=True)
    eq1 = (logits == m1).astype(jnp.float32)
    li = lax.broadcasted_iota(jnp.int32, (E, E), 0)
    lj = lax.broadcasted_iota(jnp.int32, (E, E), 1)
    incl = (li <= lj).astype(jnp.float32)
    c1 = lax.dot_general(eq1, incl, (((1,), (0,)), ((), ())),
                         preferred_element_type=jnp.float32,
                         precision=lax.Precision.HIGHEST)
    oh1 = eq1 * (c1 == 1.0).astype(jnp.float32)
    logits2 = logits - oh1 * 1e30
    m2 = jnp.max(logits2, axis=1, keepdims=True)
    eq2 = (logits2 == m2).astype(jnp.float32)
    c2 = lax.dot_general(eq2, incl, (((1,), (0,)), ((), ())),
                         preferred_element_type=jnp.float32,
                         precision=lax.Precision.HIGHEST)
    oh2 = eq2 * (c2 == 1.0).astype(jnp.float32)

    w0 = 1.0 / (1.0 + jnp.exp(m2 - m1))
    w1v = 1.0 - w0

    P = jnp.concatenate([oh1, oh2], axis=1)
    si = lax.broadcasted_iota(jnp.int32, (SB, SB), 0)
    sj = lax.broadcasted_iota(jnp.int32, (SB, SB), 1)
    tri = (sj < si).astype(jnp.bfloat16)
    carry = jnp.zeros((1, 2 * E), jnp.float32)
    rank_blocks = []
    for b in range(T // SB):
        pb = lax.slice(P, (b * SB, 0), ((b + 1) * SB, 2 * E))
        local = lax.dot_general(tri, pb.astype(jnp.bfloat16),
                                (((1,), (0,)), ((), ())),
                                preferred_element_type=jnp.float32)
        rank_blocks.append(local + carry)
        carry = carry + jnp.sum(pb, axis=0, keepdims=True)
    ranks = jnp.concatenate(rank_blocks, axis=0)
    counts0 = lax.slice(carry, (0, 0), (1, E))
    counts1 = lax.slice(carry, (0, E), (1, 2 * E))
    counts = counts0 + counts1

    padded = jnp.floor((counts + (BM - 1)) * (1.0 / BM)) * BM
    upper = (li < lj).astype(jnp.float32)
    off = lax.dot_general(padded, upper, (((1,), (0,)), ((), ())),
                          preferred_element_type=jnp.float32,
                          precision=lax.Precision.HIGHEST)
    total = jnp.sum(padded, axis=1, keepdims=True)

    rank0 = lax.slice(ranks, (0, 0), (T, E))
    rank1 = lax.slice(ranks, (0, E), (T, 2 * E))
    dest0 = jnp.sum(oh1 * (off + rank0), axis=1)
    dest1 = jnp.sum(oh2 * (off + counts0 + rank1), axis=1)

    bs = lax.broadcasted_iota(jnp.int32, (NB, E), 0).astype(jnp.float32) * BM
    ind = jnp.logical_and(bs >= off, bs < off + padded).astype(jnp.float32)
    ei = lax.broadcasted_iota(jnp.int32, (NB, E), 1).astype(jnp.float32)
    bef = jnp.sum(ei * ind, axis=1) + 7.0 * (bs[:, 0] >= total[0, 0]).astype(
        jnp.float32)

    nact = total[0, :] * (1.0 / BM)
    d0_ref[...] = dest0.astype(jnp.int32)[None, :]
    d1_ref[...] = dest1.astype(jnp.int32)[None, :]
    be_ref[...] = jnp.pad(jnp.concatenate([bef, nact]).astype(jnp.int32),
                          (0, 128 - NB - 1))[None, :]
    wb0_ref[...] = jnp.broadcast_to(w0, (T, 128))
    wb1_ref[...] = jnp.broadcast_to(w1v, (T, 128))

    xb = x.astype(jnp.bfloat16)
    lhalf = lax.bitcast_convert_type(xb[:, :D // 2], jnp.uint16)
    rhalf = lax.bitcast_convert_type(xb[:, D // 2:], jnp.uint16)
    packed = (lhalf.astype(jnp.uint32) << 16) | rhalf.astype(jnp.uint32)
    xp_ref[...] = lax.bitcast_convert_type(packed, jnp.int32)


def _gate_route(x, gate_weight):
    return pl.pallas_call(
        _gate_route_body,
        out_shape=[
            jax.ShapeDtypeStruct((1, T), jnp.int32),
            jax.ShapeDtypeStruct((1, T), jnp.int32),
            jax.ShapeDtypeStruct((1, 128), jnp.int32),
            jax.ShapeDtypeStruct((T, 128), jnp.float32),
            jax.ShapeDtypeStruct((T, 128), jnp.float32),
            jax.ShapeDtypeStruct((T, D // 2), jnp.int32),
        ],
    )(x, gate_weight)


def _dispatch_body(x_hbm, d0_hbm, d1_hbm, w0_hbm, w1_hbm, xs_hbm, ws_hbm,
                   idx_v, rows_v, wrow_v):
    wid = lax.axis_index("s") * 2 + lax.axis_index("c")
    base = wid * CHD
    pltpu.sync_copy(x_hbm.at[pl.ds(base, CHD)], rows_v)
    pltpu.sync_copy(d0_hbm.at[pl.ds(base, CHD)], idx_v)
    pltpu.sync_copy(w0_hbm.at[pl.ds(base, CHD)], wrow_v)
    pltpu.sync_copy(rows_v, xs_hbm.at[idx_v])
    pltpu.sync_copy(wrow_v, ws_hbm.at[idx_v])
    pltpu.sync_copy(d1_hbm.at[pl.ds(base, CHD)], idx_v)
    pltpu.sync_copy(w1_hbm.at[pl.ds(base, CHD)], wrow_v)
    pltpu.sync_copy(rows_v, xs_hbm.at[idx_v])
    pltpu.sync_copy(wrow_v, ws_hbm.at[idx_v])


@functools.cache
def _dispatch_kernel():
    mesh = plsc.VectorSubcoreMesh(core_axis_name="c", subcore_axis_name="s")
    return pl.kernel(
        _dispatch_body,
        mesh=mesh,
        out_type=[
            jax.ShapeDtypeStruct((CAP, D // 2), jnp.int32),
            jax.ShapeDtypeStruct((CAP, 128), jnp.float32),
        ],
        scratch_types=[
            pltpu.VMEM((CHD,), jnp.int32),
            pltpu.VMEM((CHD, D // 2), jnp.int32),
            pltpu.VMEM((CHD, 128), jnp.float32),
        ],
    )


NSPLIT = 2
FFC = FF // NSPLIT


def _gemm_body(be_ref, xs_ref, ws_ref, w1g_ref, w1u_ref, w2_ref, out_ref,
               acc_ref):
    k = pl.program_id(1)

    @pl.when(pl.program_id(0) < be_ref[NB])
    def _():
        pu = lax.bitcast_convert_type(xs_ref[...], jnp.uint32)
        lhalf = lax.bitcast_convert_type((pu >> 16).astype(jnp.uint16),
                                         jnp.bfloat16)
        rhalf = lax.bitcast_convert_type(pu.astype(jnp.uint16), jnp.bfloat16)
        xb = jnp.concatenate([lhalf, rhalf], axis=1)
        g = lax.dot_general(xb, w1g_ref[0].astype(jnp.bfloat16),
                            (((1,), (0,)), ((), ())),
                            preferred_element_type=jnp.float32)
        u = lax.dot_general(xb, w1u_ref[0].astype(jnp.bfloat16),
                            (((1,), (0,)), ((), ())),
                            preferred_element_type=jnp.float32)
        h = g * (1.0 / (1.0 + jnp.exp(-g))) * u
        h = h * ws_ref[...][:, 0:1]
        part = lax.dot_general(h.astype(jnp.bfloat16),
                               w2_ref[0].astype(jnp.bfloat16),
                               (((1,), (0,)), ((), ())),
                               preferred_element_type=jnp.float32)

        @pl.when(k == 0)
        def _():
            acc_ref[...] = part

        @pl.when(k == NSPLIT - 1)
        def _():
            hb = (acc_ref[...] + part).astype(jnp.bfloat16)
            lo = lax.bitcast_convert_type(hb[:, :D // 2], jnp.uint16)
            ro = lax.bitcast_convert_type(hb[:, D // 2:], jnp.uint16)
            out_ref[...] = lax.bitcast_convert_type(
                (lo.astype(jnp.uint32) << 16) | ro.astype(jnp.uint32),
                jnp.int32)


def _gemm(be, xs, ws, w1, w2):
    grid_spec = pltpu.PrefetchScalarGridSpec(
        num_scalar_prefetch=1,
        grid=(NB, NSPLIT),
        in_specs=[
            pl.BlockSpec(
                (BM, D // 2),
                lambda j, k, be_ref: (jnp.minimum(j, be_ref[NB] - 1), 0)),
            pl.BlockSpec(
                (BM, 128),
                lambda j, k, be_ref: (jnp.minimum(j, be_ref[NB] - 1), 0)),
            pl.BlockSpec((1, D, FFC), lambda j, k, be_ref: (be_ref[j], 0, k)),
            pl.BlockSpec((1, D, FFC),
                         lambda j, k, be_ref: (be_ref[j], 0, NSPLIT + k)),
            pl.BlockSpec((1, FFC, D), lambda j, k, be_ref: (be_ref[j], k, 0)),
        ],
        out_specs=pl.BlockSpec((BM, D // 2), lambda j, k, be_ref: (j, 0)),
        scratch_shapes=[pltpu.VMEM((BM, D), jnp.float32)],
    )
    return pl.pallas_call(
        _gemm_body,
        grid_spec=grid_spec,
        out_shape=jax.ShapeDtypeStruct((CAP, D // 2), jnp.int32),
    )(be, xs, ws, w1, w1, w2)


def _combine_body(h2_hbm, d0_hbm, d1_hbm, y_hbm, idx_v, g0_v, g1_v, y_v):
    wid = lax.axis_index("s") * 2 + lax.axis_index("c")

    @pl.loop(0, (T // NWORK) // CHC)
    def _(cstep):
        base = wid * (T // NWORK) + cstep * CHC
        pltpu.sync_copy(d0_hbm.at[pl.ds(base, CHC)], idx_v)
        pltpu.sync_copy(h2_hbm.at[idx_v], g0_v)
        pltpu.sync_copy(d1_hbm.at[pl.ds(base, CHC)], idx_v)
        pltpu.sync_copy(h2_hbm.at[idx_v], g1_v)

        @pl.loop(0, CHC)
        def _(r):
            @plsc.parallel_loop(0, D // 32, unroll=4)
            def _(c):
                sl = pl.ds(c * 16, 16)
                sh = pl.ds(D // 2 + c * 16, 16)
                p0 = g0_v[r, sl]
                p1 = g1_v[r, sl]
                l0 = plsc.bitcast(p0 & jnp.int32(-65536), jnp.float32)
                l1 = plsc.bitcast(p1 & jnp.int32(-65536), jnp.float32)
                r0 = plsc.bitcast(p0 << 16, jnp.float32)
                r1 = plsc.bitcast(p1 << 16, jnp.float32)
                y_v[r, sl] = l0 + l1
                y_v[r, sh] = r0 + r1

        pltpu.sync_copy(y_v, y_hbm.at[pl.ds(base, CHC)])


@functools.cache
def _combine_kernel():
    mesh = plsc.VectorSubcoreMesh(core_axis_name="c", subcore_axis_name="s")
    cp = pltpu.CompilerParams()
    if "needs_layout_passes" in pltpu.CompilerParams.__dataclass_fields__:
        cp = dataclasses.replace(cp, needs_layout_passes=False)
    return pl.kernel(
        _combine_body,
        mesh=mesh,
        compiler_params=cp,
        out_type=jax.ShapeDtypeStruct((T, D), jnp.float32),
        scratch_types=[
            pltpu.VMEM((CHC,), jnp.int32),
            pltpu.VMEM((CHC, D // 2), jnp.int32),
            pltpu.VMEM((CHC, D // 2), jnp.int32),
            pltpu.VMEM((CHC, D), jnp.float32),
        ],
    )


def kernel(hidden_states, gate_weight, w1, w2):
    x = hidden_states.reshape(T, D)
    d0o, d1o, beo, wb0, wb1, xp = _gate_route(x, gate_weight)
    d0 = d0o.reshape(T)
    d1 = d1o.reshape(T)
    be = beo.reshape(128)
    xs, ws = _dispatch_kernel()(xp, d0, d1, wb0, wb1)
    h2 = _gemm(be, xs, ws, w1, w2)
    y = _combine_kernel()(h2, d0, d1)
    return y.reshape(hidden_states.shape)


# speedup vs baseline: 1.3828x; 1.3828x over previous
import dataclasses
import functools

import jax
import jax.numpy as jnp
from jax import lax
from jax.experimental import pallas as pl
from jax.experimental.pallas import tpu as pltpu
from jax.experimental.pallas import tpu_sc as plsc

T = 2048
D = 768
E = 8
FF = 512
BM = 512
CAP = 8192
NB = CAP // BM
SB = 512

NWORK = 32
CHD = 64
CHC = 32


def _gate_route_body(x_ref, gw_ref, d0_ref, d1_ref, be_ref, wb0_ref, wb1_ref,
                     xp_ref):
    x = x_ref[...]
    gw = gw_ref[...]
    logits = lax.dot_general(x, gw, (((1,), (1,)), ((), ())),
                             preferred_element_type=jnp.float32)

    m1 = jnp.max(logits, axis=1, keepdims=True)
    eq1 = (logits == m1).astype(jnp.float32)
    li = lax.broadcasted_iota(jnp.int32, (E, E), 0)
    lj = lax.broadcasted_iota(jnp.int32, (E, E), 1)
    incl = (li <= lj).astype(jnp.float32)
    c1 = lax.dot_general(eq1, incl, (((1,), (0,)), ((), ())),
                         preferred_element_type=jnp.float32,
                         precision=lax.Precision.HIGHEST)
    oh1 = eq1 * (c1 == 1.0).astype(jnp.float32)
    logits2 = logits - oh1 * 1e30
    m2 = jnp.max(logits2, axis=1, keepdims=True)
    eq2 = (logits2 == m2).astype(jnp.float32)
    c2 = lax.dot_general(eq2, incl, (((1,), (0,)), ((), ())),
                         preferred_element_type=jnp.float32,
                         precision=lax.Precision.HIGHEST)
    oh2 = eq2 * (c2 == 1.0).astype(jnp.float32)

    w0 = 1.0 / (1.0 + jnp.exp(m2 - m1))
    w1v = 1.0 - w0

    P = jnp.concatenate([oh1, oh2], axis=1)
    si = lax.broadcasted_iota(jnp.int32, (SB, SB), 0)
    sj = lax.broadcasted_iota(jnp.int32, (SB, SB), 1)
    tri = (sj < si).astype(jnp.bfloat16)
    carry = jnp.zeros((1, 2 * E), jnp.float32)
    rank_blocks = []
    for b in range(T // SB):
        pb = lax.slice(P, (b * SB, 0), ((b + 1) * SB, 2 * E))
        local = lax.dot_general(tri, pb.astype(jnp.bfloat16),
                                (((1,), (0,)), ((), ())),
                                preferred_element_type=jnp.float32)
        rank_blocks.append(local + carry)
        carry = carry + jnp.sum(pb, axis=0, keepdims=True)
    ranks = jnp.concatenate(rank_blocks, axis=0)
    counts0 = lax.slice(carry, (0, 0), (1, E))
    counts1 = lax.slice(carry, (0, E), (1, 2 * E))
    counts = counts0 + counts1

    padded = jnp.floor((counts + (BM - 1)) * (1.0 / BM)) * BM
    upper = (li < lj).astype(jnp.float32)
    off = lax.dot_general(padded, upper, (((1,), (0,)), ((), ())),
                          preferred_element_type=jnp.float32,
                          precision=lax.Precision.HIGHEST)
    total = jnp.sum(padded, axis=1, keepdims=True)

    rank0 = lax.slice(ranks, (0, 0), (T, E))
    rank1 = lax.slice(ranks, (0, E), (T, 2 * E))
    dest0 = jnp.sum(oh1 * (off + rank0), axis=1)
    dest1 = jnp.sum(oh2 * (off + counts0 + rank1), axis=1)

    bs = lax.broadcasted_iota(jnp.int32, (NB, E), 0).astype(jnp.float32) * BM
    ind = jnp.logical_and(bs >= off, bs < off + padded).astype(jnp.float32)
    ei = lax.broadcasted_iota(jnp.int32, (NB, E), 1).astype(jnp.float32)
    bef = jnp.sum(ei * ind, axis=1) + 7.0 * (bs[:, 0] >= total[0, 0]).astype(
        jnp.float32)

    nact = total[0, :] * (1.0 / BM)
    d0_ref[...] = dest0.astype(jnp.int32)[None, :]
    d1_ref[...] = dest1.astype(jnp.int32)[None, :]
    be_ref[...] = jnp.pad(jnp.concatenate([bef, nact]).astype(jnp.int32),
                          (0, 128 - NB - 1))[None, :]
    wb0_ref[...] = jnp.broadcast_to(w0, (T, 128))
    wb1_ref[...] = jnp.broadcast_to(w1v, (T, 128))

    xb = x.astype(jnp.bfloat16)
    lhalf = lax.bitcast_convert_type(xb[:, :D // 2], jnp.uint16)
    rhalf = lax.bitcast_convert_type(xb[:, D // 2:], jnp.uint16)
    packed = (lhalf.astype(jnp.uint32) << 16) | rhalf.astype(jnp.uint32)
    xp_ref[...] = lax.bitcast_convert_type(packed, jnp.int32)


def _gate_route(x, gate_weight):
    return pl.pallas_call(
        _gate_route_body,
        out_shape=[
            jax.ShapeDtypeStruct((1, T), jnp.int32),
            jax.ShapeDtypeStruct((1, T), jnp.int32),
            jax.ShapeDtypeStruct((1, 128), jnp.int32),
            jax.ShapeDtypeStruct((T, 128), jnp.float32),
            jax.ShapeDtypeStruct((T, 128), jnp.float32),
            jax.ShapeDtypeStruct((T, D // 2), jnp.int32),
        ],
    )(x, gate_weight)


def _dispatch_body(x_hbm, d0_hbm, d1_hbm, w0_hbm, w1_hbm, xs_hbm, ws_hbm,
                   idx_v, rows_v, wrow_v):
    wid = lax.axis_index("s") * 2 + lax.axis_index("c")
    base = wid * CHD
    pltpu.sync_copy(x_hbm.at[pl.ds(base, CHD)], rows_v)
    pltpu.sync_copy(d0_hbm.at[pl.ds(base, CHD)], idx_v)
    pltpu.sync_copy(w0_hbm.at[pl.ds(base, CHD)], wrow_v)
    pltpu.sync_copy(rows_v, xs_hbm.at[idx_v])
    pltpu.sync_copy(wrow_v, ws_hbm.at[idx_v])
    pltpu.sync_copy(d1_hbm.at[pl.ds(base, CHD)], idx_v)
    pltpu.sync_copy(w1_hbm.at[pl.ds(base, CHD)], wrow_v)
    pltpu.sync_copy(rows_v, xs_hbm.at[idx_v])
    pltpu.sync_copy(wrow_v, ws_hbm.at[idx_v])


@functools.cache
def _dispatch_kernel():
    mesh = plsc.VectorSubcoreMesh(core_axis_name="c", subcore_axis_name="s")
    return pl.kernel(
        _dispatch_body,
        mesh=mesh,
        out_type=[
            jax.ShapeDtypeStruct((CAP, D // 2), jnp.int32),
            jax.ShapeDtypeStruct((CAP, 128), jnp.float32),
        ],
        scratch_types=[
            pltpu.VMEM((CHD,), jnp.int32),
            pltpu.VMEM((CHD, D // 2), jnp.int32),
            pltpu.VMEM((CHD, 128), jnp.float32),
        ],
    )


def _gemm_body(be_ref, xs_ref, ws_ref, w1_ref, w2_ref, out_ref):
    @pl.when(pl.program_id(0) < be_ref[NB])
    def _():
        pu = lax.bitcast_convert_type(xs_ref[...], jnp.uint32)
        lhalf = lax.bitcast_convert_type((pu >> 16).astype(jnp.uint16),
                                         jnp.bfloat16)
        rhalf = lax.bitcast_convert_type(pu.astype(jnp.uint16), jnp.bfloat16)
        xb = jnp.concatenate([lhalf, rhalf], axis=1)
        w1e = w1_ref[0].astype(jnp.bfloat16)
        fc1 = lax.dot_general(xb, w1e, (((1,), (0,)), ((), ())),
                              preferred_element_type=jnp.float32)
        g = fc1[:, :FF]
        u = fc1[:, FF:]
        h = g * (1.0 / (1.0 + jnp.exp(-g))) * u
        h = h * ws_ref[...][:, 0:1]
        w2e = w2_ref[0].astype(jnp.bfloat16)
        h2 = lax.dot_general(h.astype(jnp.bfloat16), w2e,
                             (((1,), (0,)), ((), ())),
                             preferred_element_type=jnp.float32)
        hb = h2.astype(jnp.bfloat16)
        lo = lax.bitcast_convert_type(hb[:, :D // 2], jnp.uint16)
        ro = lax.bitcast_convert_type(hb[:, D // 2:], jnp.uint16)
        out_ref[...] = lax.bitcast_convert_type(
            (lo.astype(jnp.uint32) << 16) | ro.astype(jnp.uint32), jnp.int32)


def _gemm(be, xs, ws, w1, w2):
    grid_spec = pltpu.PrefetchScalarGridSpec(
        num_scalar_prefetch=1,
        grid=(NB,),
        in_specs=[
            pl.BlockSpec((BM, D // 2),
                         lambda j, be_ref: (jnp.minimum(j, be_ref[NB] - 1), 0)),
            pl.BlockSpec((BM, 128),
                         lambda j, be_ref: (jnp.minimum(j, be_ref[NB] - 1), 0)),
            pl.BlockSpec((1, D, 2 * FF), lambda j, be_ref: (be_ref[j], 0, 0)),
            pl.BlockSpec((1, FF, D), lambda j, be_ref: (be_ref[j], 0, 0)),
        ],
        out_specs=pl.BlockSpec((BM, D // 2), lambda j, be_ref: (j, 0)),
    )
    return pl.pallas_call(
        _gemm_body,
        grid_spec=grid_spec,
        out_shape=jax.ShapeDtypeStruct((CAP, D // 2), jnp.int32),
    )(be, xs, ws, w1, w2)


def _combine_body(h2_hbm, d0_hbm, d1_hbm, y_hbm, idx_v, g0_v, g1_v, y_v):
    wid = lax.axis_index("s") * 2 + lax.axis_index("c")

    @pl.loop(0, (T // NWORK) // CHC)
    def _(cstep):
        base = wid * (T // NWORK) + cstep * CHC
        pltpu.sync_copy(d0_hbm.at[pl.ds(base, CHC)], idx_v)
        pltpu.sync_copy(h2_hbm.at[idx_v], g0_v)
        pltpu.sync_copy(d1_hbm.at[pl.ds(base, CHC)], idx_v)
        pltpu.sync_copy(h2_hbm.at[idx_v], g1_v)

        @pl.loop(0, CHC)
        def _(r):
            @plsc.parallel_loop(0, D // 32, unroll=4)
            def _(c):
                sl = pl.ds(c * 16, 16)
                sh = pl.ds(D // 2 + c * 16, 16)
                p0 = g0_v[r, sl]
                p1 = g1_v[r, sl]
                l0 = plsc.bitcast(p0 & jnp.int32(-65536), jnp.float32)
                l1 = plsc.bitcast(p1 & jnp.int32(-65536), jnp.float32)
                r0 = plsc.bitcast(p0 << 16, jnp.float32)
                r1 = plsc.bitcast(p1 << 16, jnp.float32)
                y_v[r, sl] = l0 + l1
                y_v[r, sh] = r0 + r1

        pltpu.sync_copy(y_v, y_hbm.at[pl.ds(base, CHC)])


@functools.cache
def _combine_kernel():
    mesh = plsc.VectorSubcoreMesh(core_axis_name="c", subcore_axis_name="s")
    cp = pltpu.CompilerParams()
    if "needs_layout_passes" in pltpu.CompilerParams.__dataclass_fields__:
        cp = dataclasses.replace(cp, needs_layout_passes=False)
    return pl.kernel(
        _combine_body,
        mesh=mesh,
        compiler_params=cp,
        out_type=jax.ShapeDtypeStruct((T, D), jnp.float32),
        scratch_types=[
            pltpu.VMEM((CHC,), jnp.int32),
            pltpu.VMEM((CHC, D // 2), jnp.int32),
            pltpu.VMEM((CHC, D // 2), jnp.int32),
            pltpu.VMEM((CHC, D), jnp.float32),
        ],
    )


def kernel(hidden_states, gate_weight, w1, w2):
    x = hidden_states.reshape(T, D)
    d0o, d1o, beo, wb0, wb1, xp = _gate_route(x, gate_weight)
    d0 = d0o.reshape(T)
    d1 = d1o.reshape(T)
    be = beo.reshape(128)
    xs, ws = _dispatch_kernel()(xp, d0, d1, wb0, wb1)
    h2 = _gemm(be, xs, ws, w1, w2)
    y = _combine_kernel()(h2, d0, d1)
    return y.reshape(hidden_states.shape)


# speedup vs baseline: 1.4908x; 1.0781x over previous
import dataclasses
import functools

import jax
import jax.numpy as jnp
from jax import lax
from jax.experimental import pallas as pl
from jax.experimental.pallas import tpu as pltpu
from jax.experimental.pallas import tpu_sc as plsc

T = 2048
D = 768
E = 8
FF = 512
BM = 512
CAP = 8192
NB = CAP // BM
SB = 512

NWORK = 32
CHD = 64
CHC = 32


def _gate_route_body(x_ref, gw_ref, d01_ref, be_ref, wb0_ref, wb1_ref,
                     xp_ref):
    x = x_ref[...]
    gw = gw_ref[...]
    logits = lax.dot_general(x, gw, (((1,), (1,)), ((), ())),
                             preferred_element_type=jnp.float32)

    m1 = jnp.max(logits, axis=1, keepdims=True)
    eq1 = (logits == m1).astype(jnp.float32)
    li = lax.broadcasted_iota(jnp.int32, (E, E), 0)
    lj = lax.broadcasted_iota(jnp.int32, (E, E), 1)
    incl = (li <= lj).astype(jnp.float32)
    c1 = lax.dot_general(eq1, incl, (((1,), (0,)), ((), ())),
                         preferred_element_type=jnp.float32,
                         precision=lax.Precision.HIGHEST)
    oh1 = eq1 * (c1 == 1.0).astype(jnp.float32)
    logits2 = logits - oh1 * 1e30
    m2 = jnp.max(logits2, axis=1, keepdims=True)
    eq2 = (logits2 == m2).astype(jnp.float32)
    c2 = lax.dot_general(eq2, incl, (((1,), (0,)), ((), ())),
                         preferred_element_type=jnp.float32,
                         precision=lax.Precision.HIGHEST)
    oh2 = eq2 * (c2 == 1.0).astype(jnp.float32)

    w0 = 1.0 / (1.0 + jnp.exp(m2 - m1))
    w1v = 1.0 - w0

    P = jnp.concatenate([oh1, oh2], axis=1)
    si = lax.broadcasted_iota(jnp.int32, (SB, SB), 0)
    sj = lax.broadcasted_iota(jnp.int32, (SB, SB), 1)
    tri = (sj < si).astype(jnp.bfloat16)
    carry = jnp.zeros((1, 2 * E), jnp.float32)
    rank_blocks = []
    for b in range(T // SB):
        pb = lax.slice(P, (b * SB, 0), ((b + 1) * SB, 2 * E))
        local = lax.dot_general(tri, pb.astype(jnp.bfloat16),
                                (((1,), (0,)), ((), ())),
                                preferred_element_type=jnp.float32)
        rank_blocks.append(local + carry)
        carry = carry + jnp.sum(pb, axis=0, keepdims=True)
    ranks = jnp.concatenate(rank_blocks, axis=0)
    counts0 = lax.slice(carry, (0, 0), (1, E))
    counts1 = lax.slice(carry, (0, E), (1, 2 * E))
    counts = counts0 + counts1

    padded = jnp.floor((counts + (BM - 1)) * (1.0 / BM)) * BM
    upper = (li < lj).astype(jnp.float32)
    off = lax.dot_general(padded, upper, (((1,), (0,)), ((), ())),
                          preferred_element_type=jnp.float32,
                          precision=lax.Precision.HIGHEST)
    total = jnp.sum(padded, axis=1, keepdims=True)

    rank0 = lax.slice(ranks, (0, 0), (T, E))
    rank1 = lax.slice(ranks, (0, E), (T, 2 * E))
    dest0 = jnp.sum(oh1 * (off + rank0), axis=1)
    dest1 = jnp.sum(oh2 * (off + counts0 + rank1), axis=1)

    bs = lax.broadcasted_iota(jnp.int32, (NB, E), 0).astype(jnp.float32) * BM
    ind = jnp.logical_and(bs >= off, bs < off + padded).astype(jnp.float32)
    ei = lax.broadcasted_iota(jnp.int32, (NB, E), 1).astype(jnp.float32)
    bef = jnp.sum(ei * ind, axis=1) + 7.0 * (bs[:, 0] >= total[0, 0]).astype(
        jnp.float32)

    nact = total[0, :] * (1.0 / BM)
    d01 = dest0.astype(jnp.int32) | (dest1.astype(jnp.int32) << 16)
    d01_ref[...] = d01.reshape(16, T // 16)
    be_ref[...] = jnp.pad(jnp.concatenate([bef, nact]).astype(jnp.int32),
                          (0, 128 - NB - 1))[None, :]
    wb0_ref[...] = jnp.broadcast_to(w0, (T, 128))
    wb1_ref[...] = jnp.broadcast_to(w1v, (T, 128))

    xb = x.astype(jnp.bfloat16)
    lhalf = lax.bitcast_convert_type(xb[:, :D // 2], jnp.uint16)
    rhalf = lax.bitcast_convert_type(xb[:, D // 2:], jnp.uint16)
    packed = (lhalf.astype(jnp.uint32) << 16) | rhalf.astype(jnp.uint32)
    xp_ref[...] = lax.bitcast_convert_type(packed, jnp.int32)


def _gate_route(x, gate_weight):
    return pl.pallas_call(
        _gate_route_body,
        out_shape=[
            jax.ShapeDtypeStruct((16, T // 16), jnp.int32),
            jax.ShapeDtypeStruct((1, 128), jnp.int32),
            jax.ShapeDtypeStruct((T, 128), jnp.float32),
            jax.ShapeDtypeStruct((T, 128), jnp.float32),
            jax.ShapeDtypeStruct((T, D // 2), jnp.int32),
        ],
    )(x, gate_weight)


def _unpack_idx(dp_v, i0_v, i1_v, n):
    @plsc.parallel_loop(0, n // 16)
    def _(c):
        sl = pl.ds(c * 16, 16)
        p = dp_v[sl]
        i0_v[sl] = p & jnp.int32(0xFFFF)
        i1_v[sl] = p >> 16


def _dispatch_body(x_hbm, d01_hbm, w0_hbm, w1_hbm, xs_hbm, ws_hbm,
                   dp_v, i0_v, i1_v, rows_v, wrow_v):
    wid = lax.axis_index("s") * 2 + lax.axis_index("c")
    base = wid * CHD
    pltpu.sync_copy(x_hbm.at[pl.ds(base, CHD)], rows_v)
    pltpu.sync_copy(d01_hbm.at[pl.ds(base, CHD)], dp_v)
    _unpack_idx(dp_v, i0_v, i1_v, CHD)
    pltpu.sync_copy(w0_hbm.at[pl.ds(base, CHD)], wrow_v)
    pltpu.sync_copy(rows_v, xs_hbm.at[i0_v])
    pltpu.sync_copy(wrow_v, ws_hbm.at[i0_v])
    pltpu.sync_copy(w1_hbm.at[pl.ds(base, CHD)], wrow_v)
    pltpu.sync_copy(rows_v, xs_hbm.at[i1_v])
    pltpu.sync_copy(wrow_v, ws_hbm.at[i1_v])


@functools.cache
def _dispatch_kernel():
    mesh = plsc.VectorSubcoreMesh(core_axis_name="c", subcore_axis_name="s")
    cp = pltpu.CompilerParams()
    if "needs_layout_passes" in pltpu.CompilerParams.__dataclass_fields__:
        cp = dataclasses.replace(cp, needs_layout_passes=False)
    return pl.kernel(
        _dispatch_body,
        mesh=mesh,
        compiler_params=cp,
        out_type=[
            jax.ShapeDtypeStruct((CAP, D // 2), jnp.int32),
            jax.ShapeDtypeStruct((CAP, 128), jnp.float32),
        ],
        scratch_types=[
            pltpu.VMEM((CHD,), jnp.int32),
            pltpu.VMEM((CHD,), jnp.int32),
            pltpu.VMEM((CHD,), jnp.int32),
            pltpu.VMEM((CHD, D // 2), jnp.int32),
            pltpu.VMEM((CHD, 128), jnp.float32),
        ],
    )


def _gemm_body(be_ref, xs_ref, ws_ref, w1_ref, w2_ref, out_ref):
    @pl.when(pl.program_id(0) < be_ref[NB])
    def _():
        pu = lax.bitcast_convert_type(xs_ref[...], jnp.uint32)
        lhalf = lax.bitcast_convert_type((pu >> 16).astype(jnp.uint16),
                                         jnp.bfloat16)
        rhalf = lax.bitcast_convert_type(pu.astype(jnp.uint16), jnp.bfloat16)
        xb = jnp.concatenate([lhalf, rhalf], axis=1)
        w1e = w1_ref[0].astype(jnp.bfloat16)
        fc1 = lax.dot_general(xb, w1e, (((1,), (0,)), ((), ())),
                              preferred_element_type=jnp.float32)
        g = fc1[:, :FF]
        u = fc1[:, FF:]
        h = g * (1.0 / (1.0 + jnp.exp(-g))) * u
        h = h * ws_ref[...][:, 0:1]
        w2e = w2_ref[0].astype(jnp.bfloat16)
        h2 = lax.dot_general(h.astype(jnp.bfloat16), w2e,
                             (((1,), (0,)), ((), ())),
                             preferred_element_type=jnp.float32)
        hb = h2.astype(jnp.bfloat16)
        lo = lax.bitcast_convert_type(hb[:, :D // 2], jnp.uint16)
        ro = lax.bitcast_convert_type(hb[:, D // 2:], jnp.uint16)
        out_ref[...] = lax.bitcast_convert_type(
            (lo.astype(jnp.uint32) << 16) | ro.astype(jnp.uint32), jnp.int32)


def _gemm(be, xs, ws, w1, w2):
    grid_spec = pltpu.PrefetchScalarGridSpec(
        num_scalar_prefetch=1,
        grid=(NB,),
        in_specs=[
            pl.BlockSpec((BM, D // 2),
                         lambda j, be_ref: (jnp.minimum(j, be_ref[NB] - 1), 0)),
            pl.BlockSpec((BM, 128),
                         lambda j, be_ref: (jnp.minimum(j, be_ref[NB] - 1), 0)),
            pl.BlockSpec((1, D, 2 * FF), lambda j, be_ref: (be_ref[j], 0, 0)),
            pl.BlockSpec((1, FF, D), lambda j, be_ref: (be_ref[j], 0, 0)),
        ],
        out_specs=pl.BlockSpec((BM, D // 2), lambda j, be_ref: (j, 0)),
    )
    return pl.pallas_call(
        _gemm_body,
        grid_spec=grid_spec,
        out_shape=jax.ShapeDtypeStruct((CAP, D // 2), jnp.int32),
    )(be, xs, ws, w1, w2)


def _combine_sum(g0_v, g1_v, y_v):
    @pl.loop(0, CHC)
    def _(r):
        @plsc.parallel_loop(0, D // 32, unroll=4)
        def _(c):
            sl = pl.ds(c * 16, 16)
            sh = pl.ds(D // 2 + c * 16, 16)
            p0 = g0_v[r, sl]
            p1 = g1_v[r, sl]
            l0 = plsc.bitcast(p0 & jnp.int32(-65536), jnp.float32)
            l1 = plsc.bitcast(p1 & jnp.int32(-65536), jnp.float32)
            r0 = plsc.bitcast(p0 << 16, jnp.float32)
            r1 = plsc.bitcast(p1 << 16, jnp.float32)
            y_v[r, sl] = l0 + l1
            y_v[r, sh] = r0 + r1


def _combine_body(h2_hbm, d01_hbm, y_hbm, dp_v, i0a, i1a, i0b, i1b,
                  g0a, g1a, g0b, g1b, ya, yb, sem, semw):
    wid = lax.axis_index("s") * 2 + lax.axis_index("c")
    base = wid * (2 * CHC)

    pltpu.sync_copy(d01_hbm.at[pl.ds(base, CHC)], dp_v)
    _unpack_idx(dp_v, i0a, i1a, CHC)
    ga0 = pltpu.make_async_copy(h2_hbm.at[i0a], g0a, sem)
    ga1 = pltpu.make_async_copy(h2_hbm.at[i1a], g1a, sem)
    ga0.start()
    ga1.start()
    pltpu.sync_copy(d01_hbm.at[pl.ds(base + CHC, CHC)], dp_v)
    _unpack_idx(dp_v, i0b, i1b, CHC)
    gb0 = pltpu.make_async_copy(h2_hbm.at[i0b], g0b, sem)
    gb1 = pltpu.make_async_copy(h2_hbm.at[i1b], g1b, sem)
    gb0.start()
    gb1.start()
    ga0.wait()
    ga1.wait()
    _combine_sum(g0a, g1a, ya)
    wa = pltpu.make_async_copy(ya, y_hbm.at[pl.ds(base, CHC)], semw)
    wa.start()
    gb0.wait()
    gb1.wait()
    _combine_sum(g0b, g1b, yb)
    wa.wait()
    pltpu.sync_copy(yb, y_hbm.at[pl.ds(base + CHC, CHC)])


@functools.cache
def _combine_kernel():
    mesh = plsc.VectorSubcoreMesh(core_axis_name="c", subcore_axis_name="s")
    cp = pltpu.CompilerParams()
    if "needs_layout_passes" in pltpu.CompilerParams.__dataclass_fields__:
        cp = dataclasses.replace(cp, needs_layout_passes=False)
    return pl.kernel(
        _combine_body,
        mesh=mesh,
        compiler_params=cp,
        out_type=jax.ShapeDtypeStruct((T, D), jnp.float32),
        scratch_types=[
            pltpu.VMEM((CHC,), jnp.int32),
            pltpu.VMEM((CHC,), jnp.int32),
            pltpu.VMEM((CHC,), jnp.int32),
            pltpu.VMEM((CHC,), jnp.int32),
            pltpu.VMEM((CHC,), jnp.int32),
            pltpu.VMEM((CHC, D // 2), jnp.int32),
            pltpu.VMEM((CHC, D // 2), jnp.int32),
            pltpu.VMEM((CHC, D // 2), jnp.int32),
            pltpu.VMEM((CHC, D // 2), jnp.int32),
            pltpu.VMEM((CHC, D), jnp.float32),
            pltpu.VMEM((CHC, D), jnp.float32),
            pltpu.SemaphoreType.DMA,
            pltpu.SemaphoreType.DMA,
        ],
    )


def kernel(hidden_states, gate_weight, w1, w2):
    x = hidden_states.reshape(T, D)
    d01o, beo, wb0, wb1, xp = _gate_route(x, gate_weight)
    d01 = d01o.reshape(T)
    be = beo.reshape(128)
    xs, ws = _dispatch_kernel()(xp, d01, wb0, wb1)
    h2 = _gemm(be, xs, ws, w1, w2)
    y = _combine_kernel()(h2, d01)
    return y.reshape(hidden_states.shape)
